# Initial kernel scaffold; baseline (speedup 1.0000x reference)
import jax
import jax.numpy as jnp
from jax.experimental import pallas as pl


def kernel(x, indices, norm1_g, norm1_b, w_qkv, w_proj, b_proj, norm2_g, norm2_b, w_fc1, b_fc1, w_dw, w_fc2, b_fc2):
    raise NotImplementedError("write your pallas kernel here")


# speedup vs baseline: 1.0462x; 1.0462x over previous
import functools
import math

import jax
import jax.numpy as jnp
from jax import lax
from jax.experimental import pallas as pl
from jax.experimental.pallas import tpu as pltpu
from jax.experimental.pallas import tpu_sc as plsc

B, C, H, W = 4, 192, 64, 64
N = H * W
BN = B * N
K = 2000
KP = 2048
BKP = B * KP
HID = 768
NH = 3
HD = C // NH
DUMP = BN
YROWS = BN + 512
CP = 256


def _ln1_body(x_ref, g_ref, b_ref, y_ref):
    x = x_ref[...]
    g = g_ref[0:1, :]
    b = b_ref[0:1, :]
    m = jnp.mean(x, axis=1, keepdims=True)
    v = jnp.mean((x - m) * (x - m), axis=1, keepdims=True)
    ln = (x - m) * lax.rsqrt(v + 1e-5) * g + b
    y_ref[...] = jnp.concatenate(
        [ln, jnp.zeros((ln.shape[0], CP - C), jnp.float32)], axis=1)


def _ln1(xt, g8, b8):
    grid = YROWS // 512
    return pl.pallas_call(
        _ln1_body,
        grid=(grid,),
        in_specs=[
            pl.BlockSpec((512, C), lambda i: (jnp.minimum(i, BN // 512 - 1), 0)),
            pl.BlockSpec((8, C), lambda i: (0, 0)),
            pl.BlockSpec((8, C), lambda i: (0, 0)),
        ],
        out_specs=pl.BlockSpec((512, CP), lambda i: (i, 0)),
        out_shape=jax.ShapeDtypeStruct((YROWS, CP), jnp.float32),
    )(xt, g8, b8)


def _sc_gather(y, gidx):
    info = plsc.get_sparse_core_info()
    nw = info.num_cores * info.num_subcores
    rpw = BKP // nw
    mesh = plsc.VectorSubcoreMesh(core_axis_name="c", subcore_axis_name="s")

    @functools.partial(
        pl.kernel,
        mesh=mesh,
        out_type=jax.ShapeDtypeStruct((BKP, CP), jnp.float32),
        scratch_types=[
            pltpu.VMEM((2, 128), jnp.int32),
            pltpu.VMEM((rpw, CP), jnp.float32),
            pltpu.SemaphoreType.DMA,
        ],
    )
    def k(y_hbm, idx_hbm, out_hbm, idx_v, rows_v, sem):
        wid = lax.axis_index("s") * info.num_cores + lax.axis_index("c")
        pltpu.sync_copy(idx_hbm.at[wid], idx_v)
        c0 = pltpu.async_copy(y_hbm.at[idx_v.at[0]], rows_v.at[pl.ds(0, 128)], sem)
        c1 = pltpu.async_copy(y_hbm.at[idx_v.at[1]], rows_v.at[pl.ds(128, 128)], sem)
        c0.wait()
        c1.wait()
        pltpu.sync_copy(rows_v, out_hbm.at[pl.ds(wid * rpw, rpw)])

    return k(y, gidx)


def _qkv_body(s_ref, w_ref, o_ref):
    o_ref[...] = jnp.dot(s_ref[:, :C], w_ref[...],
                         preferred_element_type=jnp.float32)


def _qkv(sel, w_qkv):
    return pl.pallas_call(
        _qkv_body,
        grid=(BKP // 512,),
        in_specs=[
            pl.BlockSpec((512, CP), lambda i: (i, 0)),
            pl.BlockSpec((C, 3 * C), lambda i: (0, 0)),
        ],
        out_specs=pl.BlockSpec((512, 3 * C), lambda i: (i, 0)),
        out_shape=jax.ShapeDtypeStruct((BKP, 3 * C), jnp.float32),
    )(sel, w_qkv)


QT = 256


def _attn_body(q_ref, kv_ref, wp_ref, bp_ref, o_ref):
    scale = HD ** -0.5
    colmask = jnp.where(
        lax.broadcasted_iota(jnp.int32, (1, KP), 1) >= K, -1e30, 0.0)
    outs = []
    for h in range(NH):
        q = q_ref[:, h * HD:(h + 1) * HD]
        kk = kv_ref[:, C + h * HD:C + (h + 1) * HD]
        v = kv_ref[:, 2 * C + h * HD:2 * C + (h + 1) * HD]
        s = lax.dot_general(q, kk, (((1,), (1,)), ((), ())),
                            preferred_element_type=jnp.float32)
        s = s * scale + colmask
        s = s - jnp.max(s, axis=1, keepdims=True)
        e = jnp.exp(s)
        p = e / jnp.sum(e, axis=1, keepdims=True)
        outs.append(jnp.dot(p, v, preferred_element_type=jnp.float32))
    o = jnp.concatenate(outs, axis=1)
    proj = (jnp.dot(o, wp_ref[...], preferred_element_type=jnp.float32)
            + bp_ref[0:1, :])
    o_ref[...] = jnp.concatenate(
        [proj, jnp.zeros((proj.shape[0], CP - C), jnp.float32)], axis=1)


def _attn(qkv, w_proj, bp8):
    nt = KP // QT
    return pl.pallas_call(
        _attn_body,
        grid=(B, nt),
        in_specs=[
            pl.BlockSpec((QT, 3 * C), lambda b, t: (b * nt + t, 0)),
            pl.BlockSpec((KP, 3 * C), lambda b, t: (b, 0)),
            pl.BlockSpec((C, C), lambda b, t: (0, 0)),
            pl.BlockSpec((8, C), lambda b, t: (0, 0)),
        ],
        out_specs=pl.BlockSpec((QT, CP), lambda b, t: (b * nt + t, 0)),
        out_shape=jax.ShapeDtypeStruct((BKP, CP), jnp.float32),
    )(qkv, qkv, w_proj, bp8)


def _sc_scatter(y, op, sidx):
    info = plsc.get_sparse_core_info()
    nw = info.num_subcores
    rpw = BKP // nw
    cpw = BN // nw
    mesh = plsc.VectorSubcoreMesh(
        core_axis_name="c", subcore_axis_name="s", num_cores=1)

    @functools.partial(
        pl.kernel,
        mesh=mesh,
        out_type=jax.ShapeDtypeStruct((YROWS, CP), jnp.float32),
        scratch_types=[
            pltpu.VMEM((4, 128), jnp.int32),
            pltpu.VMEM((256, CP), jnp.float32),
            pltpu.SemaphoreType.DMA,
        ],
    )
    def k(y_hbm, o_hbm, idx_hbm, out_hbm, idx_v, rows_v, sem):
        wid = lax.axis_index("s")
        pltpu.sync_copy(y_hbm.at[pl.ds(wid * cpw, cpw)],
                        out_hbm.at[pl.ds(wid * cpw, cpw)])
        plsc.subcore_barrier()
        pltpu.sync_copy(idx_hbm.at[wid], idx_v)
        for half in range(2):
            pltpu.sync_copy(
                o_hbm.at[pl.ds(wid * rpw + half * 256, 256)], rows_v)
            copies = [
                pltpu.async_copy(rows_v.at[pl.ds(j * 128, 128)],
                                 out_hbm.at[idx_v.at[half * 2 + j]], sem)
                for j in range(2)
            ]
            for cp in copies:
                cp.wait()

    return k(y, op, sidx)


def _mlp1_body(x_ref, y2_ref, g_ref, b_ref, w1_ref, b1_ref, zn_ref, h1_ref):
    z = x_ref[...] + y2_ref[:, :C]
    m = jnp.mean(z, axis=1, keepdims=True)
    v = jnp.mean((z - m) * (z - m), axis=1, keepdims=True)
    zn = (z - m) * lax.rsqrt(v + 1e-5) * g_ref[0:1, :] + b_ref[0:1, :]
    zn_ref[...] = zn
    h1_ref[...] = (jnp.dot(zn, w1_ref[...], preferred_element_type=jnp.float32)
                   + b1_ref[0:1, :])


def _mlp1(xt, y2, g8, b8, w_fc1, b18):
    return pl.pallas_call(
        _mlp1_body,
        grid=(BN // 512,),
        in_specs=[
            pl.BlockSpec((512, C), lambda i: (i, 0)),
            pl.BlockSpec((512, CP), lambda i: (i, 0)),
            pl.BlockSpec((8, C), lambda i: (0, 0)),
            pl.BlockSpec((8, C), lambda i: (0, 0)),
            pl.BlockSpec((C, HID), lambda i: (0, 0)),
            pl.BlockSpec((8, HID), lambda i: (0, 0)),
        ],
        out_specs=[
            pl.BlockSpec((512, C), lambda i: (i, 0)),
            pl.BlockSpec((512, HID), lambda i: (i, 0)),
        ],
        out_shape=[
            jax.ShapeDtypeStruct((BN, C), jnp.float32),
            jax.ShapeDtypeStruct((BN, HID), jnp.float32),
        ],
    )(xt, y2, g8, b8, w_fc1, b18)


RT = 1024


def _gelu(x):
    p = 0.3275911
    a1, a2, a3, a4, a5 = (0.254829592, -0.284496736, 1.421413741,
                          -1.453152027, 1.061405429)
    z = x * (2.0 ** -0.5)
    s = jnp.sign(z)
    az = jnp.abs(z)
    t = 1.0 / (1.0 + p * az)
    poly = ((((a5 * t + a4) * t + a3) * t + a2) * t + a1) * t
    erf = s * (1.0 - poly * jnp.exp(-az * az))
    return 0.5 * x * (1.0 + erf)


def _conv_body(cur_ref, prv_ref, nxt_ref, wd_ref, zn_ref, w2_ref, b2_ref,
               o_ref):
    blk = pl.program_id(1)
    zrow = jnp.zeros((8, HID), jnp.float32)
    ext = jnp.concatenate(
        [zrow, prv_ref[...], cur_ref[...], nxt_ref[...], zrow], axis=0)
    wpos = lax.broadcasted_iota(jnp.int32, (RT, 1), 0) % W
    grow = blk * 16 + lax.broadcasted_iota(jnp.int32, (RT, 1), 0) // W
    acc = jnp.zeros((RT, HID), jnp.float32)
    for a in range(3):
        for b in range(3):
            di, dj = a - 1, b - 1
            sl = ext[72 + di * 64 + dj: 72 + di * 64 + dj + RT, :]
            m = jnp.ones((RT, 1), jnp.bool_)
            if dj == -1:
                m = jnp.logical_and(m, wpos > 0)
            if dj == 1:
                m = jnp.logical_and(m, wpos < W - 1)
            if di == -1:
                m = jnp.logical_and(m, grow > 0)
            if di == 1:
                m = jnp.logical_and(m, grow < H - 1)
            w_row = wd_ref[a * 3 + b: a * 3 + b + 1, :]
            acc = acc + sl * w_row * m.astype(jnp.float32)
    gel = _gelu(acc)
    o_ref[...] = (zn_ref[...]
                  + jnp.dot(gel, w2_ref[...], preferred_element_type=jnp.float32)
                  + b2_ref[0:1, :])


def _conv_mlp2(h1, wd16, zn, w_fc2, b28):
    nb = N // RT
    upb = N // 64

    def cur_map(bb, t):
        return (bb * nb + t, 0)

    def prv_map(bb, t):
        return (jnp.maximum(bb * upb, (bb * nb + t) * 16 - 1), 0)

    def nxt_map(bb, t):
        return (jnp.minimum(bb * upb + upb - 1, (bb * nb + t) * 16 + 16), 0)

    return pl.pallas_call(
        _conv_body,
        grid=(B, nb),
        in_specs=[
            pl.BlockSpec((RT, HID), cur_map),
            pl.BlockSpec((64, HID), prv_map),
            pl.BlockSpec((64, HID), nxt_map),
            pl.BlockSpec((16, HID), lambda bb, t: (0, 0)),
            pl.BlockSpec((RT, C), cur_map),
            pl.BlockSpec((HID, C), lambda bb, t: (0, 0)),
            pl.BlockSpec((8, C), lambda bb, t: (0, 0)),
        ],
        out_specs=pl.BlockSpec((RT, C), cur_map),
        out_shape=jax.ShapeDtypeStruct((BN, C), jnp.float32),
    )(h1, h1, h1, wd16, zn, w_fc2, b28)


@jax.jit
def kernel(x, indices, norm1_g, norm1_b, w_qkv, w_proj, b_proj, norm2_g,
           norm2_b, w_fc1, b_fc1, w_dw, w_fc2, b_fc2):
    xt = jnp.transpose(x.reshape(B, C, N), (0, 2, 1)).reshape(BN, C)

    idx = indices.astype(jnp.int32) + (jnp.arange(B, dtype=jnp.int32) * N)[:, None]
    idx = jnp.concatenate(
        [idx, jnp.full((B, KP - K), DUMP, jnp.int32)], axis=1).reshape(-1)
    gidx = idx.reshape(32, 2, 128)
    sidx = idx.reshape(16, 4, 128)

    def r8(v):
        return jnp.broadcast_to(v[None, :], (8, v.shape[0]))

    g1, b1 = r8(norm1_g), r8(norm1_b)
    g2, b2 = r8(norm2_g), r8(norm2_b)
    bp8, b18, b28 = r8(b_proj), r8(b_fc1), r8(b_fc2)
    wd16 = jnp.concatenate(
        [w_dw.reshape(HID, 9).T, jnp.zeros((7, HID), jnp.float32)], axis=0)

    y = _ln1(xt, g1, b1)
    sel = _sc_gather(y, gidx)
    qkv = _qkv(sel, w_qkv)
    op = _attn(qkv, w_proj, bp8)
    y2 = _sc_scatter(y, op, sidx)
    zn, h1 = _mlp1(xt, y2[:BN], g2, b2, w_fc1, b18)
    out = _conv_mlp2(h1, wd16, zn, w_fc2, b28)
    return jnp.transpose(out.reshape(B, N, C), (0, 2, 1)).reshape(B, C, H, W)
